# baseline (speedup 1.0000x reference)
import functools

import jax
import jax.numpy as jnp
import numpy as np
from jax import lax
from jax.experimental import pallas as pl
from jax.experimental.pallas import tpu as pltpu
from jax.experimental.pallas import tpu_sc as plsc

N = 50000
E = 100000
R = 6
IN = 121
H = 128
B = 64

NPAD = 51200
NTILES = 16
EPT = E // NTILES
TE = 6272
NB_A = TE // 128
CHUNK = 1024
NCHUNK = NPAD // CHUNK
RPT = CHUNK // NTILES
CAP = 6400
DUMP = CAP - 16
BN = 512

f32 = jnp.float32
i32 = jnp.int32


def _k0_body(x_ref, psel_ref, pos_ref, o_ref):
    xb = x_ref[...]
    pidx = jnp.clip(xb[:, 5:6].astype(i32), 0, 23)
    oh = (pidx == lax.broadcasted_iota(i32, (1, 24), 1)).astype(f32)
    o_ref[...] = (jnp.dot(xb, psel_ref[...], preferred_element_type=f32)
                  + jnp.dot(oh, pos_ref[...], preferred_element_type=f32))


def _assemble(xp, psel, pos_pad):
    return pl.pallas_call(
        _k0_body,
        grid=(NPAD // BN,),
        in_specs=[
            pl.BlockSpec((BN, H), lambda i: (i, 0)),
            pl.BlockSpec((H, H), lambda i: (0, 0)),
            pl.BlockSpec((24, H), lambda i: (0, 0)),
        ],
        out_specs=pl.BlockSpec((BN, H), lambda i: (i, 0)),
        out_shape=jax.ShapeDtypeStruct((NPAD, H), f32),
    )(xp, psel, pos_pad)


def _k1_body(h_ref, w_ref, as_ref, ad_ref, hs_ref, av_ref, dv_ref):
    hb = h_ref[...]
    hsb = jnp.dot(hb, w_ref[0], preferred_element_type=f32)
    hs_ref[0] = hsb
    av_ref[0, 0] = jnp.sum(hsb * as_ref[0], axis=1)
    dv_ref[0, 0] = jnp.sum(hsb * ad_ref[0], axis=1)


def _project(h, W, a_s, a_d):
    return pl.pallas_call(
        _k1_body,
        grid=(NPAD // BN, R),
        in_specs=[
            pl.BlockSpec((BN, H), lambda i, r: (i, 0)),
            pl.BlockSpec((1, H, H), lambda i, r: (r, 0, 0)),
            pl.BlockSpec((1, 1, H), lambda i, r: (r, 0, 0)),
            pl.BlockSpec((1, 1, H), lambda i, r: (r, 0, 0)),
        ],
        out_specs=[
            pl.BlockSpec((1, BN, H), lambda i, r: (r, i, 0)),
            pl.BlockSpec((1, 1, BN), lambda i, r: (r, 0, i)),
            pl.BlockSpec((1, 1, BN), lambda i, r: (r, 0, i)),
        ],
        out_shape=[
            jax.ShapeDtypeStruct((R, NPAD, H), f32),
            jax.ShapeDtypeStruct((R, 1, NPAD), f32),
            jax.ShapeDtypeStruct((R, 1, NPAD), f32),
        ],
    )(h, W, a_s.reshape(R, 1, H), a_d.reshape(R, 1, H))


def _gelu_ln(hb, g_ref, be_ref):
    hb = 0.5 * hb * (1.0 + lax.erf(hb * 0.7071067811865476))
    mu = jnp.mean(hb, axis=1, keepdims=True)
    d = hb - mu
    v = jnp.mean(d * d, axis=1, keepdims=True)
    return d * lax.rsqrt(v + 1e-5) * g_ref[...] + be_ref[...]


def _k2_body(p_ref, bias_ref, g_ref, be_ref, o_ref):
    hb = p_ref[0] + p_ref[1] + bias_ref[...]
    o_ref[...] = _gelu_ln(hb, g_ref, be_ref)


def _k2s_body(p_ref, bias_ref, g_ref, be_ref, q_ref, o_ref, s_ref):
    hb = p_ref[0] + p_ref[1] + bias_ref[...]
    hn = _gelu_ln(hb, g_ref, be_ref)
    o_ref[...] = hn
    s_ref[...] = jnp.sum(hn * q_ref[...], axis=1, keepdims=True)


def _post(parts, bias_sum, g, be, query=None):
    base_in = [
        pl.BlockSpec((2, BN, H), lambda i: (0, i, 0)),
        pl.BlockSpec((1, H), lambda i: (0, 0)),
        pl.BlockSpec((1, H), lambda i: (0, 0)),
        pl.BlockSpec((1, H), lambda i: (0, 0)),
    ]
    if query is None:
        return pl.pallas_call(
            _k2_body,
            grid=(NPAD // BN,),
            in_specs=base_in,
            out_specs=pl.BlockSpec((BN, H), lambda i: (i, 0)),
            out_shape=jax.ShapeDtypeStruct((NPAD, H), f32),
        )(parts, bias_sum, g, be)
    return pl.pallas_call(
        _k2s_body,
        grid=(NPAD // BN,),
        in_specs=base_in + [pl.BlockSpec((1, H), lambda i: (0, 0))],
        out_specs=[
            pl.BlockSpec((BN, H), lambda i: (i, 0)),
            pl.BlockSpec((BN, 1), lambda i: (i, 0)),
        ],
        out_shape=[
            jax.ShapeDtypeStruct((NPAD, H), f32),
            jax.ShapeDtypeStruct((NPAD, 1), f32),
        ],
    )(parts, bias_sum, g, be, query)


def _k3_body(h_ref, b_ref, m_ref, q_ref, wp_ref, bp_ref, o_ref, accn, accd):
    i = pl.program_id(0)

    @pl.when(i == 0)
    def _():
        accn[...] = jnp.zeros_like(accn)
        accd[...] = jnp.zeros_like(accd)

    hb = h_ref[...]
    s2 = jnp.sum(hb * q_ref[...], axis=1, keepdims=True)
    e = jnp.exp(s2 - m_ref[0, 0])
    ohT = (lax.broadcasted_iota(i32, (B, 1), 0) == b_ref[...]).astype(f32)
    accn[...] += jnp.dot(ohT, e * hb, preferred_element_type=f32)
    accd[...] += jnp.dot(ohT, jnp.broadcast_to(e, (BN, H)),
                         preferred_element_type=f32)

    @pl.when(i == NPAD // BN - 1)
    def _():
        pool = accn[...] / (accd[...] + 1e-16)
        o_ref[...] = jnp.dot(pool, wp_ref[...], preferred_element_type=f32) + bp_ref[...]


def _pool(h, batch2d, M, query, Wp, bp):
    return pl.pallas_call(
        _k3_body,
        grid=(NPAD // BN,),
        in_specs=[
            pl.BlockSpec((BN, H), lambda i: (i, 0)),
            pl.BlockSpec((1, BN), lambda i: (0, i)),
            pl.BlockSpec(memory_space=pltpu.SMEM),
            pl.BlockSpec((1, H), lambda i: (0, 0)),
            pl.BlockSpec((H, H), lambda i: (0, 0)),
            pl.BlockSpec((1, H), lambda i: (0, 0)),
        ],
        out_specs=pl.BlockSpec((B, H), lambda i: (0, 0)),
        out_shape=jax.ShapeDtypeStruct((B, H), f32),
        scratch_shapes=[pltpu.VMEM((B, H), f32), pltpu.VMEM((B, H), f32)],
    )(h, batch2d, M, query, Wp, bp)


_MESH = plsc.VectorSubcoreMesh(core_axis_name="c", subcore_axis_name="s",
                               num_cores=2, num_subcores=NTILES)


def _sc_body(edges, hs, av, dv, cvec, out, den_o,
             srcb, dstb, wb, gbuf, praw, cv,
             csrc, cdst, cwb, idxst, rows, zb1, zb2, sem, sem2, den_s, acc_s):
    c = lax.axis_index("c")
    s = lax.axis_index("s")
    zv = jnp.zeros((16,), f32)
    ziv = jnp.zeros((16,), i32)
    iota16 = lax.iota(i32, 16)
    colv = [(q * 16 + iota16) for q in range(8)]
    mo = lambda v: pl.multiple_of(v, 8)

    def _z1(i, _):
        zb1[pl.ds(i * 16, 16)] = zv
        return 0
    lax.fori_loop(0, 1600 // 16, _z1, 0)

    def _z2(i, _):
        for q in range(8):
            zb2[i, pl.ds(q * 16, 16)] = zv
        return 0
    lax.fori_loop(0, 32, _z2, 0)

    for j in range(3):
        r = c * 3 + j
        pltpu.sync_copy(edges.at[pl.ds(mo(((r * 2 + 0) * NTILES + s) * TE), TE)],
                        srcb.at[0, pl.ds(j * TE, TE)])
        pltpu.sync_copy(edges.at[pl.ds(mo(((r * 2 + 1) * NTILES + s) * TE), TE)],
                        dstb.at[0, pl.ds(j * TE, TE)])
        off = r * NPAD

        def _adj(i, _, j=j, off=off):
            sl = pl.ds(j * TE + i * 16, 16)
            srcb[0, sl] = srcb[0, sl] + off
            dstb[0, sl] = dstb[0, sl] + off
            return 0
        lax.fori_loop(0, TE // 16, _adj, 0)

    GB = 3 * TE // 7

    def _gath(b, _):
        sl = pl.ds(pl.multiple_of(b * GB, 128), GB)
        cp1 = pltpu.async_copy(av.at[srcb.at[0, sl]], wb.at[0, sl], sem)
        cp2 = pltpu.async_copy(dv.at[dstb.at[0, sl]], gbuf.at[0, sl], sem2)
        cp1.wait()
        cp2.wait()
        return 0
    lax.fori_loop(0, 7, _gath, 0)
    NSL = NPAD // NTILES
    BLK = TE // 7
    for j in range(3):
        r = c * 3 + j
        def _zd(i, _):
            pltpu.sync_copy(zb1, den_s.at[pl.ds(mo(s * NSL + i * 1600), 1600)])
            return 0
        lax.fori_loop(0, 2, _zd, 0)
        plsc.subcore_barrier()
        pltpu.sync_copy(cvec.at[r], cv)
        cvv = cv[...]

        def _ev(i, _, j=j, cvv=cvv):
            sl = pl.ds(j * TE + i * 16, 16)
            al = wb[0, sl] + gbuf[0, sl]
            al = jnp.where(al > 0, al, 0.2 * al)
            ev = jnp.exp(al - cvv)
            vid = i * 16 + iota16
            wb[0, sl] = jnp.where(vid < EPT, ev, 0.0)
            return 0
        lax.fori_loop(0, TE // 16, _ev, 0)

        def _sadd(b, _, j=j, r=r):
            def _loc(i, _):
                praw[0, pl.ds(i * 16, 16)] = (
                    dstb[0, pl.ds(j * TE + b * BLK + i * 16, 16)] - r * NPAD)
                return 0
            lax.fori_loop(0, BLK // 16, _loc, 0)
            sl = pl.ds(pl.multiple_of(j * TE + b * BLK, 128), BLK)
            pltpu.sync_copy(wb.at[0, sl], den_s.at[praw.at[0]], add=True)
            return 0
        lax.fori_loop(0, 7, _sadd, 0)
        plsc.subcore_barrier()
        pltpu.sync_copy(den_s.at[pl.ds(mo(s * NSL), NSL)],
                        den_o.at[pl.ds(mo(r * NPAD + s * NSL), NSL)])
        plsc.subcore_barrier()

    def _dget(b, _):
        sl = pl.ds(pl.multiple_of(b * GB, 128), GB)
        pltpu.async_copy(den_o.at[dstb.at[0, sl]], gbuf.at[0, sl], sem).wait()
        return 0
    lax.fori_loop(0, 7, _dget, 0)

    def _dv(i, _):
        sl = pl.ds(i * 16, 16)
        wb[0, sl] = wb[0, sl] / (gbuf[0, sl] + 1e-16)
        return 0
    lax.fori_loop(0, 3 * TE // 16, _dv, 0)

    def _chunk(ch, _):
        lo = ch * CHUNK

        def _zacc(i, _):
            pltpu.sync_copy(zb2, acc_s.at[pl.ds(mo(s * RPT + i * 32), 32), :])
            return 0
        lax.fori_loop(0, RPT // 32, _zacc, 0)
        plsc.subcore_barrier()

        def _rel(j2, _, lo=lo):
            r = c * 3 + j2
            cbase = r * NPAD + lo

            def _scan(b, ptr, j2=j2, cbase=cbase):
                for k in range(8):
                    sl = pl.ds(j2 * TE + b * 128 + k * 16, 16)
                    d = dstb[0, sl]
                    m = (d >= cbase) & (d < cbase + CHUNK)
                    vid = b * 128 + k * 16 + iota16
                    m = m & (vid < EPT)
                    pos = plsc.cumsum(m.astype(i32))
                    off = jnp.where(m, ptr + pos - 1, DUMP + iota16)
                    plsc.store_scatter(cdst, [off], d - cbase)
                    plsc.store_scatter(csrc, [off], srcb[0, sl])
                    plsc.store_scatter(cwb, [off], wb[0, sl])
                    ptr = ptr + pos[15]
                return ptr
            ptr = lax.fori_loop(0, NB_A, _scan, jnp.int32(0))

            for k in range(8):
                psl = pl.ds(ptr + k * 16, 16)
                cdst[psl] = ziv
                csrc[psl] = ziv
                cwb[psl] = zv
            nb = (ptr + 127) // 128

            def _gsc(g, _):
                base = g * 128
                for k in range(8):
                    idxst[0, pl.ds(k * 16, 16)] = csrc[pl.ds(base + k * 16, 16)]
                pltpu.async_copy(hs.at[idxst.at[0]], rows, sem).wait()
                for k in range(8):
                    idxst[0, pl.ds(k * 16, 16)] = cdst[pl.ds(base + k * 16, 16)]

                def _scale(e2, _, base=base):
                    rsp = jnp.full((16,), e2, i32)
                    wsp = plsc.load_gather(cwb, [jnp.full((16,), base + e2, i32)])
                    for q in range(8):
                        v = plsc.load_gather(rows, [rsp, colv[q]])
                        plsc.store_scatter(rows, [rsp, colv[q]], v * wsp)
                    return 0
                lax.fori_loop(0, 128, _scale, 0)
                pltpu.sync_copy(rows, acc_s.at[idxst.at[0]], add=True)
                return 0
            lax.fori_loop(0, nb, _gsc, 0)
            return 0
        lax.fori_loop(0, 3, _rel, 0)
        plsc.subcore_barrier()

        def _wout(i, _, lo=lo):
            a = s * RPT + i * 32
            pltpu.sync_copy(acc_s.at[pl.ds(mo(a), 32), :],
                            out.at[c, pl.ds(mo(lo + a), 32), :])
            return 0
        lax.fori_loop(0, RPT // 32, _wout, 0)
        plsc.subcore_barrier()
        return 0
    lax.fori_loop(0, NCHUNK, _chunk, 0)


@functools.partial(
    pl.kernel,
    out_type=(jax.ShapeDtypeStruct((2, NPAD, H), f32),
              jax.ShapeDtypeStruct((R * NPAD,), f32)),
    mesh=_MESH,
    compiler_params=pltpu.CompilerParams(needs_layout_passes=False),
    scratch_types=[
        pltpu.VMEM((1, 3 * TE), i32),
        pltpu.VMEM((1, 3 * TE), i32),
        pltpu.VMEM((1, 3 * TE), f32),
        pltpu.VMEM((1, 3 * TE), f32),
        pltpu.VMEM((1, TE // 7), i32),
        pltpu.VMEM((16,), f32),
        pltpu.VMEM((CAP,), i32),
        pltpu.VMEM((CAP,), i32),
        pltpu.VMEM((CAP,), f32),
        pltpu.VMEM((1, 128), i32),
        pltpu.VMEM((128, H), f32),
        pltpu.VMEM((1600,), f32),
        pltpu.VMEM((32, H), f32),
        pltpu.SemaphoreType.DMA,
        pltpu.SemaphoreType.DMA,
        pltpu.VMEM_SHARED((NPAD,), f32),
        pltpu.VMEM_SHARED((CHUNK, H), f32),
    ],
)
def _sc_edge(edges, hs, av, dv, cvec, out, den_o, *scratch):
    _sc_body(edges, hs, av, dv, cvec, out, den_o, *scratch)


def _gat_layer(h, edges_t, W, a_s, a_d):
    hs, av, dv = _project(h, W, a_s, a_d)
    av2 = av.reshape(R, NPAD)
    dv2 = dv.reshape(R, NPAD)
    z = av2.max(axis=1) + dv2.max(axis=1)
    Cr = jnp.where(z > 0, z, 0.2 * z)
    cvec = jnp.tile(Cr[:, None], (1, 16))
    parts, _ = _sc_edge(edges_t.reshape(-1), hs.reshape(R * NPAD, H),
                        av2.reshape(-1), dv2.reshape(-1), cvec)
    return parts


_SEL = np.zeros((H, H), np.float32)
for _j in range(120):
    _SEL[_j if _j < 5 else _j + 1, _j] = 1.0


def kernel(x, edge_index, batch, pos_emb, W1, a_src1, a_dst1, b1,
           W2, a_src2, a_dst2, b2, g1, be1, g2, be2, query, Wp, bp):
    xp = jnp.zeros((NPAD, H), f32).at[:N, :IN].set(x)
    batch2d = jnp.full((1, NPAD), B, i32).at[0, :N].set(batch)
    edges_t = (jnp.zeros((R, 2, NTILES, TE), i32)
               .at[:, :, :, :EPT].set(edge_index.reshape(R, 2, NTILES, EPT)))
    psel = jnp.asarray(_SEL)
    pos_pad = jnp.zeros((24, H), f32).at[:, 120:].set(pos_emb)
    g1r = g1.reshape(1, H)
    be1r = be1.reshape(1, H)
    g2r = g2.reshape(1, H)
    be2r = be2.reshape(1, H)
    qr = query.reshape(1, H)
    bpr = bp.reshape(1, H)
    bs1 = b1.sum(axis=0).reshape(1, H)
    bs2 = b2.sum(axis=0).reshape(1, H)

    h = _assemble(xp, psel, pos_pad)
    parts1 = _gat_layer(h, edges_t, W1, a_src1, a_dst1)
    h = _post(parts1, bs1, g1r, be1r)
    parts2 = _gat_layer(h, edges_t, W2, a_src2, a_dst2)
    h, scores = _post(parts2, bs2, g2r, be2r, query=qr)
    M = jnp.max(scores).reshape(1, 1)
    return _pool(h, batch2d, M, qr, Wp, bpr)


# speedup vs baseline: 1.8693x; 1.8693x over previous
import functools

import jax
import jax.numpy as jnp
import numpy as np
from jax import lax
from jax.experimental import pallas as pl
from jax.experimental.pallas import tpu as pltpu
from jax.experimental.pallas import tpu_sc as plsc

N = 50000
E = 100000
R = 6
IN = 121
H = 128
B = 64

NPAD = 51200
NTILES = 16
EPT = E // NTILES
TE = 6272
NB_A = TE // 128
CHUNK = 3200
NCHUNK = NPAD // CHUNK
RPT = CHUNK // NTILES
CAP = 6400
DUMP = CAP - 16
BN = 512

f32 = jnp.float32
i32 = jnp.int32


def _k0_body(x_ref, psel_ref, pos_ref, o_ref):
    xb = x_ref[...]
    pidx = jnp.clip(xb[:, 5:6].astype(i32), 0, 23)
    oh = (pidx == lax.broadcasted_iota(i32, (1, 24), 1)).astype(f32)
    o_ref[...] = (jnp.dot(xb, psel_ref[...], preferred_element_type=f32)
                  + jnp.dot(oh, pos_ref[...], preferred_element_type=f32))


def _assemble(xp, psel, pos_pad):
    return pl.pallas_call(
        _k0_body,
        grid=(NPAD // BN,),
        in_specs=[
            pl.BlockSpec((BN, H), lambda i: (i, 0)),
            pl.BlockSpec((H, H), lambda i: (0, 0)),
            pl.BlockSpec((24, H), lambda i: (0, 0)),
        ],
        out_specs=pl.BlockSpec((BN, H), lambda i: (i, 0)),
        out_shape=jax.ShapeDtypeStruct((NPAD, H), f32),
    )(xp, psel, pos_pad)


def _k1_body(h_ref, w_ref, as_ref, ad_ref, hs_ref, av_ref, dv_ref):
    hb = h_ref[...]
    hsb = jnp.dot(hb, w_ref[0], preferred_element_type=f32)
    hs_ref[0] = hsb
    av_ref[0, 0] = jnp.sum(hsb * as_ref[0], axis=1)
    dv_ref[0, 0] = jnp.sum(hsb * ad_ref[0], axis=1)


def _project(h, W, a_s, a_d):
    return pl.pallas_call(
        _k1_body,
        grid=(NPAD // BN, R),
        in_specs=[
            pl.BlockSpec((BN, H), lambda i, r: (i, 0)),
            pl.BlockSpec((1, H, H), lambda i, r: (r, 0, 0)),
            pl.BlockSpec((1, 1, H), lambda i, r: (r, 0, 0)),
            pl.BlockSpec((1, 1, H), lambda i, r: (r, 0, 0)),
        ],
        out_specs=[
            pl.BlockSpec((1, BN, H), lambda i, r: (r, i, 0)),
            pl.BlockSpec((1, 1, BN), lambda i, r: (r, 0, i)),
            pl.BlockSpec((1, 1, BN), lambda i, r: (r, 0, i)),
        ],
        out_shape=[
            jax.ShapeDtypeStruct((R, NPAD, H), f32),
            jax.ShapeDtypeStruct((R, 1, NPAD), f32),
            jax.ShapeDtypeStruct((R, 1, NPAD), f32),
        ],
    )(h, W, a_s.reshape(R, 1, H), a_d.reshape(R, 1, H))


def _gelu_ln(hb, g_ref, be_ref):
    hb = 0.5 * hb * (1.0 + lax.erf(hb * 0.7071067811865476))
    mu = jnp.mean(hb, axis=1, keepdims=True)
    d = hb - mu
    v = jnp.mean(d * d, axis=1, keepdims=True)
    return d * lax.rsqrt(v + 1e-5) * g_ref[...] + be_ref[...]


def _k2_body(p_ref, bias_ref, g_ref, be_ref, o_ref):
    hb = p_ref[0] + p_ref[1] + bias_ref[...]
    o_ref[...] = _gelu_ln(hb, g_ref, be_ref)


def _k2s_body(p_ref, bias_ref, g_ref, be_ref, q_ref, o_ref, s_ref):
    hb = p_ref[0] + p_ref[1] + bias_ref[...]
    hn = _gelu_ln(hb, g_ref, be_ref)
    o_ref[...] = hn
    s_ref[...] = jnp.sum(hn * q_ref[...], axis=1, keepdims=True)


def _post(parts, bias_sum, g, be, query=None):
    base_in = [
        pl.BlockSpec((2, BN, H), lambda i: (0, i, 0)),
        pl.BlockSpec((1, H), lambda i: (0, 0)),
        pl.BlockSpec((1, H), lambda i: (0, 0)),
        pl.BlockSpec((1, H), lambda i: (0, 0)),
    ]
    if query is None:
        return pl.pallas_call(
            _k2_body,
            grid=(NPAD // BN,),
            in_specs=base_in,
            out_specs=pl.BlockSpec((BN, H), lambda i: (i, 0)),
            out_shape=jax.ShapeDtypeStruct((NPAD, H), f32),
        )(parts, bias_sum, g, be)
    return pl.pallas_call(
        _k2s_body,
        grid=(NPAD // BN,),
        in_specs=base_in + [pl.BlockSpec((1, H), lambda i: (0, 0))],
        out_specs=[
            pl.BlockSpec((BN, H), lambda i: (i, 0)),
            pl.BlockSpec((BN, 1), lambda i: (i, 0)),
        ],
        out_shape=[
            jax.ShapeDtypeStruct((NPAD, H), f32),
            jax.ShapeDtypeStruct((NPAD, 1), f32),
        ],
    )(parts, bias_sum, g, be, query)


def _k3_body(h_ref, b_ref, m_ref, q_ref, wp_ref, bp_ref, o_ref, accn, accd):
    i = pl.program_id(0)

    @pl.when(i == 0)
    def _():
        accn[...] = jnp.zeros_like(accn)
        accd[...] = jnp.zeros_like(accd)

    hb = h_ref[...]
    s2 = jnp.sum(hb * q_ref[...], axis=1, keepdims=True)
    e = jnp.exp(s2 - m_ref[0, 0])
    ohT = (lax.broadcasted_iota(i32, (B, 1), 0) == b_ref[...]).astype(f32)
    accn[...] += jnp.dot(ohT, e * hb, preferred_element_type=f32)
    accd[...] += jnp.dot(ohT, jnp.broadcast_to(e, (BN, H)),
                         preferred_element_type=f32)

    @pl.when(i == NPAD // BN - 1)
    def _():
        pool = accn[...] / (accd[...] + 1e-16)
        o_ref[...] = jnp.dot(pool, wp_ref[...], preferred_element_type=f32) + bp_ref[...]


def _pool(h, batch2d, M, query, Wp, bp):
    return pl.pallas_call(
        _k3_body,
        grid=(NPAD // BN,),
        in_specs=[
            pl.BlockSpec((BN, H), lambda i: (i, 0)),
            pl.BlockSpec((1, BN), lambda i: (0, i)),
            pl.BlockSpec(memory_space=pltpu.SMEM),
            pl.BlockSpec((1, H), lambda i: (0, 0)),
            pl.BlockSpec((H, H), lambda i: (0, 0)),
            pl.BlockSpec((1, H), lambda i: (0, 0)),
        ],
        out_specs=pl.BlockSpec((B, H), lambda i: (0, 0)),
        out_shape=jax.ShapeDtypeStruct((B, H), f32),
        scratch_shapes=[pltpu.VMEM((B, H), f32), pltpu.VMEM((B, H), f32)],
    )(h, batch2d, M, query, Wp, bp)


_MESH = plsc.VectorSubcoreMesh(core_axis_name="c", subcore_axis_name="s",
                               num_cores=2, num_subcores=NTILES)


def _sc_body(edges, hs, av, dv, cvec, out, den_o,
             srcb, dstb, wb, gbuf, gbuf2, praw, cv,
             csrc, cdst, cwb, idxst, pidx, rows, zb1, zb2, sem, sem2, den_s,
             acc_s):
    c = lax.axis_index("c")
    s = lax.axis_index("s")
    zv = jnp.zeros((16,), f32)
    ziv = jnp.zeros((16,), i32)
    iota16 = lax.iota(i32, 16)
    colv = [(q * 16 + iota16) for q in range(8)]
    mo = lambda v: pl.multiple_of(v, 8)

    def _z1(i, _):
        zb1[pl.ds(i * 16, 16)] = zv
        return 0
    lax.fori_loop(0, 1600 // 16, _z1, 0)

    def _z2(i, _):
        for q in range(8):
            zb2[i, pl.ds(q * 16, 16)] = zv
        return 0
    lax.fori_loop(0, 8, _z2, 0)

    for j in range(3):
        r = c * 3 + j
        pltpu.sync_copy(edges.at[pl.ds(mo(((r * 2 + 0) * NTILES + s) * TE), TE)],
                        srcb.at[pl.ds(j * TE, TE)])
        pltpu.sync_copy(edges.at[pl.ds(mo(((r * 2 + 1) * NTILES + s) * TE), TE)],
                        dstb.at[pl.ds(j * TE, TE)])
        off = r * NPAD

        def _adj(i, _, j=j, off=off):
            sl = pl.ds(j * TE + i * 16, 16)
            srcb[sl] = srcb[sl] + off
            dstb[sl] = dstb[sl] + off
            return 0
        lax.fori_loop(0, TE // 16, _adj, 0)

    NSL = NPAD // NTILES

    for j in range(3):
        r = c * 3 + j
        def _zd(i, _):
            pltpu.sync_copy(zb1, den_s.at[pl.ds(mo(s * NSL + i * 1600), 1600)])
            return 0
        lax.fori_loop(0, 2, _zd, 0)
        plsc.subcore_barrier()
        pltpu.sync_copy(cvec.at[r], cv)
        cvv = cv[...]

        def _batch_a(b, _, j=j, r=r, cvv=cvv):
            base = j * TE + b * 896

            def _st1(k, _):
                idxst[0, pl.ds(k * 16, 16)] = srcb[pl.ds(base + k * 16, 16)]
                return 0
            lax.fori_loop(0, 56, _st1, 0)
            cpa = pltpu.async_copy(av.at[idxst.at[0]], gbuf, sem)

            def _st2(k, _):
                praw[0, pl.ds(k * 16, 16)] = dstb[pl.ds(base + k * 16, 16)]
                return 0
            lax.fori_loop(0, 56, _st2, 0)
            cpb = pltpu.async_copy(dv.at[praw.at[0]], gbuf2, sem2)
            cpa.wait()
            cpb.wait()

            def _cmp(k, _, cvv=cvv):
                slk = pl.ds(k * 16, 16)
                al = gbuf[slk] + gbuf2[slk]
                al = jnp.where(al > 0, al, 0.2 * al)
                ev = jnp.exp(al - cvv)
                vid = b * 896 + k * 16 + iota16
                ev = jnp.where(vid < EPT, ev, 0.0)
                wb[pl.ds(base + k * 16, 16)] = ev
                praw[0, slk] = praw[0, slk] - r * NPAD
                return 0
            lax.fori_loop(0, 56, _cmp, 0)
            pltpu.sync_copy(wb.at[pl.ds(pl.multiple_of(base, 128), 896)],
                            den_s.at[praw.at[0]], add=True)
            return 0
        lax.fori_loop(0, TE // 896, _batch_a, 0)
        plsc.subcore_barrier()
        pltpu.sync_copy(den_s.at[pl.ds(mo(s * NSL), NSL)],
                        den_o.at[pl.ds(mo(r * NPAD + s * NSL), NSL)])
        plsc.subcore_barrier()

        def _batch_w(b, _, j=j):
            base = j * TE + b * 896

            def _st3(k, _):
                idxst[0, pl.ds(k * 16, 16)] = dstb[pl.ds(base + k * 16, 16)]
                return 0
            lax.fori_loop(0, 56, _st3, 0)
            pltpu.async_copy(den_o.at[idxst.at[0]], gbuf, sem).wait()

            def _div(k, _):
                sl = pl.ds(base + k * 16, 16)
                wb[sl] = wb[sl] / (gbuf[pl.ds(k * 16, 16)] + 1e-16)
                return 0
            lax.fori_loop(0, 56, _div, 0)
            return 0
        lax.fori_loop(0, TE // 896, _batch_w, 0)

    def _chunk(ch, _):
        lo = ch * CHUNK

        def _zacc(i, _):
            pltpu.sync_copy(zb2, acc_s.at[pl.ds(mo(s * RPT + i * 8), 8), :])
            return 0
        lax.fori_loop(0, RPT // 8, _zacc, 0)
        plsc.subcore_barrier()

        def _rel(j2, _, lo=lo):
            r = c * 3 + j2
            cbase = r * NPAD + lo

            def _scan(b, ptr, j2=j2, cbase=cbase):
                for k in range(8):
                    sl = pl.ds(j2 * TE + b * 128 + k * 16, 16)
                    d = dstb[sl]
                    m = (d >= cbase) & (d < cbase + CHUNK)
                    vid = b * 128 + k * 16 + iota16
                    m = m & (vid < EPT)
                    pos = plsc.cumsum(m.astype(i32))
                    off = jnp.where(m, ptr + pos - 1, DUMP + iota16)
                    plsc.store_scatter(cdst, [off], d - cbase)
                    plsc.store_scatter(csrc, [off], srcb[sl])
                    plsc.store_scatter(cwb, [off], wb[sl])
                    ptr = ptr + pos[15]
                return ptr
            ptr = lax.fori_loop(0, NB_A, _scan, jnp.int32(0))

            for k in range(8):
                psl = pl.ds(ptr + k * 16, 16)
                cdst[psl] = ziv
                csrc[psl] = ziv
                cwb[psl] = zv
            nb = (ptr + 127) // 128

            def _gsc(g, _):
                base = g * 128
                for k in range(8):
                    pidx[0, pl.ds(k * 16, 16)] = csrc[pl.ds(base + k * 16, 16)]
                pltpu.async_copy(hs.at[pidx.at[0]], rows, sem).wait()
                for k in range(8):
                    pidx[0, pl.ds(k * 16, 16)] = cdst[pl.ds(base + k * 16, 16)]

                def _scale(e2, _, base=base):
                    rsp = jnp.full((16,), e2, i32)
                    wsp = plsc.load_gather(cwb, [jnp.full((16,), base + e2, i32)])
                    for q in range(8):
                        v = plsc.load_gather(rows, [rsp, colv[q]])
                        plsc.store_scatter(rows, [rsp, colv[q]], v * wsp)
                    return 0
                lax.fori_loop(0, 128, _scale, 0)
                pltpu.sync_copy(rows, acc_s.at[pidx.at[0]], add=True)
                return 0
            lax.fori_loop(0, nb, _gsc, 0)
            return 0
        lax.fori_loop(0, 3, _rel, 0)
        plsc.subcore_barrier()

        def _wout(i, _, lo=lo):
            a = s * RPT + i * 8
            pltpu.sync_copy(acc_s.at[pl.ds(mo(a), 8), :],
                            out.at[c, pl.ds(mo(lo + a), 8), :])
            return 0
        lax.fori_loop(0, RPT // 8, _wout, 0)
        plsc.subcore_barrier()
        return 0
    lax.fori_loop(0, NCHUNK, _chunk, 0)


@functools.partial(
    pl.kernel,
    out_type=(jax.ShapeDtypeStruct((2, NPAD, H), f32),
              jax.ShapeDtypeStruct((R * NPAD,), f32)),
    mesh=_MESH,
    compiler_params=pltpu.CompilerParams(needs_layout_passes=False),
    scratch_types=[
        pltpu.VMEM((3 * TE,), i32),
        pltpu.VMEM((3 * TE,), i32),
        pltpu.VMEM((3 * TE,), f32),
        pltpu.VMEM((896,), f32),
        pltpu.VMEM((896,), f32),
        pltpu.VMEM((1, 896), i32),
        pltpu.VMEM((16,), f32),
        pltpu.VMEM((CAP,), i32),
        pltpu.VMEM((CAP,), i32),
        pltpu.VMEM((CAP,), f32),
        pltpu.VMEM((1, 896), i32),
        pltpu.VMEM((1, 128), i32),
        pltpu.VMEM((128, H), f32),
        pltpu.VMEM((1600,), f32),
        pltpu.VMEM((8, H), f32),
        pltpu.SemaphoreType.DMA,
        pltpu.SemaphoreType.DMA,
        pltpu.VMEM_SHARED((NPAD,), f32),
        pltpu.VMEM_SHARED((CHUNK, H), f32),
    ],
)
def _sc_edge(edges, hs, av, dv, cvec, out, den_o, *scratch):
    _sc_body(edges, hs, av, dv, cvec, out, den_o, *scratch)


def _gat_layer(h, edges_t, W, a_s, a_d):
    hs, av, dv = _project(h, W, a_s, a_d)
    av2 = av.reshape(R, NPAD)
    dv2 = dv.reshape(R, NPAD)
    z = av2.max(axis=1) + dv2.max(axis=1)
    Cr = jnp.where(z > 0, z, 0.2 * z)
    cvec = jnp.tile(Cr[:, None], (1, 16))
    parts, _ = _sc_edge(edges_t.reshape(-1), hs.reshape(R * NPAD, H),
                        av2.reshape(-1), dv2.reshape(-1), cvec)
    return parts


_SEL = np.zeros((H, H), np.float32)
for _j in range(120):
    _SEL[_j if _j < 5 else _j + 1, _j] = 1.0


def kernel(x, edge_index, batch, pos_emb, W1, a_src1, a_dst1, b1,
           W2, a_src2, a_dst2, b2, g1, be1, g2, be2, query, Wp, bp):
    xp = jnp.zeros((NPAD, H), f32).at[:N, :IN].set(x)
    batch2d = jnp.full((1, NPAD), B, i32).at[0, :N].set(batch)
    edges_t = (jnp.zeros((R, 2, NTILES, TE), i32)
               .at[:, :, :, :EPT].set(edge_index.reshape(R, 2, NTILES, EPT)))
    psel = jnp.asarray(_SEL)
    pos_pad = jnp.zeros((24, H), f32).at[:, 120:].set(pos_emb)
    g1r = g1.reshape(1, H)
    be1r = be1.reshape(1, H)
    g2r = g2.reshape(1, H)
    be2r = be2.reshape(1, H)
    qr = query.reshape(1, H)
    bpr = bp.reshape(1, H)
    bs1 = b1.sum(axis=0).reshape(1, H)
    bs2 = b2.sum(axis=0).reshape(1, H)

    h = _assemble(xp, psel, pos_pad)
    parts1 = _gat_layer(h, edges_t, W1, a_src1, a_dst1)
    h = _post(parts1, bs1, g1r, be1r)
    parts2 = _gat_layer(h, edges_t, W2, a_src2, a_dst2)
    h, scores = _post(parts2, bs2, g2r, be2r, query=qr)
    M = jnp.max(scores).reshape(1, 1)
    return _pool(h, batch2d, M, qr, Wp, bpr)


# speedup vs baseline: 1.9374x; 1.0364x over previous
import functools

import jax
import jax.numpy as jnp
import numpy as np
from jax import lax
from jax.experimental import pallas as pl
from jax.experimental.pallas import tpu as pltpu
from jax.experimental.pallas import tpu_sc as plsc

N = 50000
E = 100000
R = 6
IN = 121
H = 128
B = 64

NPAD = 51200
NTILES = 16
EPT = E // NTILES
TE = 6272
NB_A = TE // 128
CHUNK = 3200
NCHUNK = NPAD // CHUNK
RPT = CHUNK // NTILES
CAP = 6400
DUMP = CAP - 16
BN = 512

f32 = jnp.float32
i32 = jnp.int32


def _k0_body(x_ref, psel_ref, pos_ref, o_ref):
    xb = x_ref[...]
    pidx = jnp.clip(xb[:, 5:6].astype(i32), 0, 23)
    oh = (pidx == lax.broadcasted_iota(i32, (1, 24), 1)).astype(f32)
    o_ref[...] = (jnp.dot(xb, psel_ref[...], preferred_element_type=f32)
                  + jnp.dot(oh, pos_ref[...], preferred_element_type=f32))


def _assemble(xp, psel, pos_pad):
    return pl.pallas_call(
        _k0_body,
        grid=(NPAD // BN,),
        in_specs=[
            pl.BlockSpec((BN, H), lambda i: (i, 0)),
            pl.BlockSpec((H, H), lambda i: (0, 0)),
            pl.BlockSpec((24, H), lambda i: (0, 0)),
        ],
        out_specs=pl.BlockSpec((BN, H), lambda i: (i, 0)),
        out_shape=jax.ShapeDtypeStruct((NPAD, H), f32),
    )(xp, psel, pos_pad)


def _k1_body(h_ref, w_ref, as_ref, ad_ref, hs_ref, av_ref, dv_ref):
    hb = h_ref[...]
    hsb = jnp.dot(hb, w_ref[0], preferred_element_type=f32)
    hs_ref[0] = hsb
    av_ref[0, 0] = jnp.sum(hsb * as_ref[0], axis=1)
    dv_ref[0, 0] = jnp.sum(hsb * ad_ref[0], axis=1)


def _project(h, W, a_s, a_d):
    return pl.pallas_call(
        _k1_body,
        grid=(NPAD // BN, R),
        in_specs=[
            pl.BlockSpec((BN, H), lambda i, r: (i, 0)),
            pl.BlockSpec((1, H, H), lambda i, r: (r, 0, 0)),
            pl.BlockSpec((1, 1, H), lambda i, r: (r, 0, 0)),
            pl.BlockSpec((1, 1, H), lambda i, r: (r, 0, 0)),
        ],
        out_specs=[
            pl.BlockSpec((1, BN, H), lambda i, r: (r, i, 0)),
            pl.BlockSpec((1, 1, BN), lambda i, r: (r, 0, i)),
            pl.BlockSpec((1, 1, BN), lambda i, r: (r, 0, i)),
        ],
        out_shape=[
            jax.ShapeDtypeStruct((R, NPAD, H), f32),
            jax.ShapeDtypeStruct((R, 1, NPAD), f32),
            jax.ShapeDtypeStruct((R, 1, NPAD), f32),
        ],
    )(h, W, a_s.reshape(R, 1, H), a_d.reshape(R, 1, H))


def _gelu_ln(hb, g_ref, be_ref):
    hb = 0.5 * hb * (1.0 + lax.erf(hb * 0.7071067811865476))
    mu = jnp.mean(hb, axis=1, keepdims=True)
    d = hb - mu
    v = jnp.mean(d * d, axis=1, keepdims=True)
    return d * lax.rsqrt(v + 1e-5) * g_ref[...] + be_ref[...]


def _k2_body(p_ref, bias_ref, g_ref, be_ref, o_ref):
    hb = p_ref[0] + p_ref[1] + bias_ref[...]
    o_ref[...] = _gelu_ln(hb, g_ref, be_ref)


def _k2s_body(p_ref, bias_ref, g_ref, be_ref, q_ref, o_ref, s_ref):
    hb = p_ref[0] + p_ref[1] + bias_ref[...]
    hn = _gelu_ln(hb, g_ref, be_ref)
    o_ref[...] = hn
    s_ref[...] = jnp.sum(hn * q_ref[...], axis=1, keepdims=True)


def _post(parts, bias_sum, g, be, query=None):
    base_in = [
        pl.BlockSpec((2, BN, H), lambda i: (0, i, 0)),
        pl.BlockSpec((1, H), lambda i: (0, 0)),
        pl.BlockSpec((1, H), lambda i: (0, 0)),
        pl.BlockSpec((1, H), lambda i: (0, 0)),
    ]
    if query is None:
        return pl.pallas_call(
            _k2_body,
            grid=(NPAD // BN,),
            in_specs=base_in,
            out_specs=pl.BlockSpec((BN, H), lambda i: (i, 0)),
            out_shape=jax.ShapeDtypeStruct((NPAD, H), f32),
        )(parts, bias_sum, g, be)
    return pl.pallas_call(
        _k2s_body,
        grid=(NPAD // BN,),
        in_specs=base_in + [pl.BlockSpec((1, H), lambda i: (0, 0))],
        out_specs=[
            pl.BlockSpec((BN, H), lambda i: (i, 0)),
            pl.BlockSpec((BN, 1), lambda i: (i, 0)),
        ],
        out_shape=[
            jax.ShapeDtypeStruct((NPAD, H), f32),
            jax.ShapeDtypeStruct((NPAD, 1), f32),
        ],
    )(parts, bias_sum, g, be, query)


def _k3_body(h_ref, b_ref, m_ref, q_ref, wp_ref, bp_ref, o_ref, accn, accd):
    i = pl.program_id(0)

    @pl.when(i == 0)
    def _():
        accn[...] = jnp.zeros_like(accn)
        accd[...] = jnp.zeros_like(accd)

    hb = h_ref[...]
    s2 = jnp.sum(hb * q_ref[...], axis=1, keepdims=True)
    e = jnp.exp(s2 - m_ref[0, 0])
    ohT = (lax.broadcasted_iota(i32, (B, 1), 0) == b_ref[...]).astype(f32)
    accn[...] += jnp.dot(ohT, e * hb, preferred_element_type=f32)
    accd[...] += jnp.dot(ohT, jnp.broadcast_to(e, (BN, H)),
                         preferred_element_type=f32)

    @pl.when(i == NPAD // BN - 1)
    def _():
        pool = accn[...] / (accd[...] + 1e-16)
        o_ref[...] = jnp.dot(pool, wp_ref[...], preferred_element_type=f32) + bp_ref[...]


def _pool(h, batch2d, M, query, Wp, bp):
    return pl.pallas_call(
        _k3_body,
        grid=(NPAD // BN,),
        in_specs=[
            pl.BlockSpec((BN, H), lambda i: (i, 0)),
            pl.BlockSpec((1, BN), lambda i: (0, i)),
            pl.BlockSpec(memory_space=pltpu.SMEM),
            pl.BlockSpec((1, H), lambda i: (0, 0)),
            pl.BlockSpec((H, H), lambda i: (0, 0)),
            pl.BlockSpec((1, H), lambda i: (0, 0)),
        ],
        out_specs=pl.BlockSpec((B, H), lambda i: (0, 0)),
        out_shape=jax.ShapeDtypeStruct((B, H), f32),
        scratch_shapes=[pltpu.VMEM((B, H), f32), pltpu.VMEM((B, H), f32)],
    )(h, batch2d, M, query, Wp, bp)


_MESH = plsc.VectorSubcoreMesh(core_axis_name="c", subcore_axis_name="s",
                               num_cores=2, num_subcores=NTILES)


def _sc_body(edges, hs, av, dv, cvec, out, den_o,
             srcb, dstb, wb, gbuf, gbuf2, praw, cv,
             csrc, cdst, cwb, idxst, pidx, rows, zb1, zb2, sem, sem2, den_s,
             acc_s):
    c = lax.axis_index("c")
    s = lax.axis_index("s")
    zv = jnp.zeros((16,), f32)
    ziv = jnp.zeros((16,), i32)
    iota16 = lax.iota(i32, 16)
    colv = [(q * 16 + iota16) for q in range(8)]
    mo = lambda v: pl.multiple_of(v, 8)

    def _z1(i, _):
        zb1[pl.ds(i * 16, 16)] = zv
        return 0
    lax.fori_loop(0, 800 // 16, _z1, 0)

    def _z2(i, _):
        for q in range(8):
            zb2[i, pl.ds(q * 16, 16)] = zv
        return 0
    lax.fori_loop(0, 40, _z2, 0)

    for j in range(3):
        r = c * 3 + j
        pltpu.sync_copy(edges.at[pl.ds(mo(((r * 2 + 0) * NTILES + s) * TE), TE)],
                        srcb.at[pl.ds(j * TE, TE)])
        pltpu.sync_copy(edges.at[pl.ds(mo(((r * 2 + 1) * NTILES + s) * TE), TE)],
                        dstb.at[pl.ds(j * TE, TE)])
        off = r * NPAD

        def _adj(i, _, j=j, off=off):
            sl = pl.ds(j * TE + i * 16, 16)
            srcb[sl] = srcb[sl] + off
            dstb[sl] = dstb[sl] + off
            return 0
        lax.fori_loop(0, TE // 16, _adj, 0)

    NSL = NPAD // NTILES

    for j in range(3):
        r = c * 3 + j
        def _zd(i, _):
            pltpu.sync_copy(zb1, den_s.at[pl.ds(mo(s * NSL + i * 800), 800)])
            return 0
        lax.fori_loop(0, 4, _zd, 0)
        plsc.subcore_barrier()
        pltpu.sync_copy(cvec.at[r], cv)
        cvv = cv[...]

        def _batch_a(b, _, j=j, r=r, cvv=cvv):
            base = j * TE + b * 896

            def _st1(k, _):
                idxst[0, pl.ds(k * 16, 16)] = srcb[pl.ds(base + k * 16, 16)]
                return 0
            lax.fori_loop(0, 56, _st1, 0)
            cpa = pltpu.async_copy(av.at[idxst.at[0]], gbuf, sem)

            def _st2(k, _):
                praw[0, pl.ds(k * 16, 16)] = dstb[pl.ds(base + k * 16, 16)]
                return 0
            lax.fori_loop(0, 56, _st2, 0)
            cpb = pltpu.async_copy(dv.at[praw.at[0]], gbuf2, sem2)
            cpa.wait()
            cpb.wait()

            def _cmp(k, _, cvv=cvv):
                slk = pl.ds(k * 16, 16)
                al = gbuf[slk] + gbuf2[slk]
                al = jnp.where(al > 0, al, 0.2 * al)
                ev = jnp.exp(al - cvv)
                vid = b * 896 + k * 16 + iota16
                ev = jnp.where(vid < EPT, ev, 0.0)
                wb[pl.ds(base + k * 16, 16)] = ev
                praw[0, slk] = praw[0, slk] - r * NPAD
                return 0
            lax.fori_loop(0, 56, _cmp, 0)
            pltpu.sync_copy(wb.at[pl.ds(pl.multiple_of(base, 128), 896)],
                            den_s.at[praw.at[0]], add=True)
            return 0
        lax.fori_loop(0, TE // 896, _batch_a, 0)
        plsc.subcore_barrier()
        pltpu.sync_copy(den_s.at[pl.ds(mo(s * NSL), NSL)],
                        den_o.at[pl.ds(mo(r * NPAD + s * NSL), NSL)])
        plsc.subcore_barrier()

        def _batch_w(b, _, j=j):
            base = j * TE + b * 896

            def _st3(k, _):
                idxst[0, pl.ds(k * 16, 16)] = dstb[pl.ds(base + k * 16, 16)]
                return 0
            lax.fori_loop(0, 56, _st3, 0)
            pltpu.async_copy(den_o.at[idxst.at[0]], gbuf, sem).wait()

            def _div(k, _):
                sl = pl.ds(base + k * 16, 16)
                wb[sl] = wb[sl] / (gbuf[pl.ds(k * 16, 16)] + 1e-16)
                return 0
            lax.fori_loop(0, 56, _div, 0)
            return 0
        lax.fori_loop(0, TE // 896, _batch_w, 0)

    def _chunk(ch, _):
        lo = ch * CHUNK

        def _zacc(i, _):
            pltpu.sync_copy(zb2, acc_s.at[pl.ds(mo(s * RPT + i * 40), 40), :])
            return 0
        lax.fori_loop(0, RPT // 40, _zacc, 0)
        plsc.subcore_barrier()

        def _rel(j2, _, lo=lo):
            r = c * 3 + j2
            cbase = r * NPAD + lo

            def _scan(b, ptr, j2=j2, cbase=cbase):
                for k in range(8):
                    sl = pl.ds(j2 * TE + b * 128 + k * 16, 16)
                    d = dstb[sl]
                    m = (d >= cbase) & (d < cbase + CHUNK)
                    vid = b * 128 + k * 16 + iota16
                    m = m & (vid < EPT)
                    pos = plsc.cumsum(m.astype(i32))
                    off = jnp.where(m, ptr + pos - 1, DUMP + iota16)
                    plsc.store_scatter(cdst, [off], d - cbase)
                    plsc.store_scatter(csrc, [off], srcb[sl])
                    plsc.store_scatter(cwb, [off], wb[sl])
                    ptr = ptr + pos[15]
                return ptr
            ptr = lax.fori_loop(0, NB_A, _scan, jnp.int32(0))

            for k in range(8):
                psl = pl.ds(ptr + k * 16, 16)
                cdst[psl] = ziv
                csrc[psl] = ziv
                cwb[psl] = zv
            nb = (ptr + 127) // 128

            def _gsc(g, _):
                base = g * 128
                for k in range(8):
                    pidx[0, pl.ds(k * 16, 16)] = csrc[pl.ds(base + k * 16, 16)]
                pltpu.async_copy(hs.at[pidx.at[0]], rows, sem).wait()
                for k in range(8):
                    pidx[0, pl.ds(k * 16, 16)] = cdst[pl.ds(base + k * 16, 16)]

                def _scale(e2, _, base=base):
                    rsp = jnp.full((16,), e2, i32)
                    wsp = plsc.load_gather(cwb, [jnp.full((16,), base + e2, i32)])
                    for q in range(8):
                        v = plsc.load_gather(rows, [rsp, colv[q]])
                        plsc.store_scatter(rows, [rsp, colv[q]], v * wsp)
                    return 0
                lax.fori_loop(0, 128, _scale, 0)
                pltpu.sync_copy(rows, acc_s.at[pidx.at[0]], add=True)
                return 0
            lax.fori_loop(0, nb, _gsc, 0)
            return 0
        lax.fori_loop(0, 3, _rel, 0)
        plsc.subcore_barrier()

        pltpu.sync_copy(acc_s.at[pl.ds(mo(s * RPT), RPT), :],
                        out.at[c, pl.ds(mo(lo + s * RPT), RPT), :])
        plsc.subcore_barrier()
        return 0
    lax.fori_loop(0, NCHUNK, _chunk, 0)


@functools.partial(
    pl.kernel,
    out_type=(jax.ShapeDtypeStruct((2, NPAD, H), f32),
              jax.ShapeDtypeStruct((R * NPAD,), f32)),
    mesh=_MESH,
    compiler_params=pltpu.CompilerParams(needs_layout_passes=False),
    scratch_types=[
        pltpu.VMEM((3 * TE,), i32),
        pltpu.VMEM((3 * TE,), i32),
        pltpu.VMEM((3 * TE,), f32),
        pltpu.VMEM((896,), f32),
        pltpu.VMEM((896,), f32),
        pltpu.VMEM((1, 896), i32),
        pltpu.VMEM((16,), f32),
        pltpu.VMEM((CAP,), i32),
        pltpu.VMEM((CAP,), i32),
        pltpu.VMEM((CAP,), f32),
        pltpu.VMEM((1, 896), i32),
        pltpu.VMEM((1, 128), i32),
        pltpu.VMEM((128, H), f32),
        pltpu.VMEM((800,), f32),
        pltpu.VMEM((40, H), f32),
        pltpu.SemaphoreType.DMA,
        pltpu.SemaphoreType.DMA,
        pltpu.VMEM_SHARED((NPAD,), f32),
        pltpu.VMEM_SHARED((CHUNK, H), f32),
    ],
)
def _sc_edge(edges, hs, av, dv, cvec, out, den_o, *scratch):
    _sc_body(edges, hs, av, dv, cvec, out, den_o, *scratch)


def _gat_layer(h, edges_t, W, a_s, a_d):
    hs, av, dv = _project(h, W, a_s, a_d)
    av2 = av.reshape(R, NPAD)
    dv2 = dv.reshape(R, NPAD)
    z = av2.max(axis=1) + dv2.max(axis=1)
    Cr = jnp.where(z > 0, z, 0.2 * z)
    cvec = jnp.tile(Cr[:, None], (1, 16))
    parts, _ = _sc_edge(edges_t.reshape(-1), hs.reshape(R * NPAD, H),
                        av2.reshape(-1), dv2.reshape(-1), cvec)
    return parts


_SEL = np.zeros((H, H), np.float32)
for _j in range(120):
    _SEL[_j if _j < 5 else _j + 1, _j] = 1.0


def kernel(x, edge_index, batch, pos_emb, W1, a_src1, a_dst1, b1,
           W2, a_src2, a_dst2, b2, g1, be1, g2, be2, query, Wp, bp):
    xp = jnp.zeros((NPAD, H), f32).at[:N, :IN].set(x)
    batch2d = jnp.full((1, NPAD), B, i32).at[0, :N].set(batch)
    edges_t = (jnp.zeros((R, 2, NTILES, TE), i32)
               .at[:, :, :, :EPT].set(edge_index.reshape(R, 2, NTILES, EPT)))
    psel = jnp.asarray(_SEL)
    pos_pad = jnp.zeros((24, H), f32).at[:, 120:].set(pos_emb)
    g1r = g1.reshape(1, H)
    be1r = be1.reshape(1, H)
    g2r = g2.reshape(1, H)
    be2r = be2.reshape(1, H)
    qr = query.reshape(1, H)
    bpr = bp.reshape(1, H)
    bs1 = b1.sum(axis=0).reshape(1, H)
    bs2 = b2.sum(axis=0).reshape(1, H)

    h = _assemble(xp, psel, pos_pad)
    parts1 = _gat_layer(h, edges_t, W1, a_src1, a_dst1)
    h = _post(parts1, bs1, g1r, be1r)
    parts2 = _gat_layer(h, edges_t, W2, a_src2, a_dst2)
    h, scores = _post(parts2, bs2, g2r, be2r, query=qr)
    M = jnp.max(scores).reshape(1, 1)
    return _pool(h, batch2d, M, qr, Wp, bpr)


# speedup vs baseline: 3.5761x; 1.8459x over previous
import functools

import jax
import jax.numpy as jnp
import numpy as np
from jax import lax
from jax.experimental import pallas as pl
from jax.experimental.pallas import tpu as pltpu
from jax.experimental.pallas import tpu_sc as plsc

N = 50000
E = 100000
R = 6
IN = 121
H = 128
B = 64

NPAD = 51200
NTILES = 16
EPT = E // NTILES
TE = 6272
NB_A = TE // 128
CHUNK = 3200
NCHUNK = NPAD // CHUNK
RPT = CHUNK // NTILES
CAP = 6400
DUMP = CAP - 16
BN = 512

f32 = jnp.float32
i32 = jnp.int32


def _k0_body(x_ref, psel_ref, pos_ref, o_ref):
    xb = x_ref[...]
    pidx = jnp.clip(xb[:, 5:6].astype(i32), 0, 23)
    oh = (pidx == lax.broadcasted_iota(i32, (1, 24), 1)).astype(f32)
    o_ref[...] = (jnp.dot(xb, psel_ref[...], preferred_element_type=f32)
                  + jnp.dot(oh, pos_ref[...], preferred_element_type=f32))


def _assemble(xp, psel, pos_pad):
    return pl.pallas_call(
        _k0_body,
        grid=(NPAD // BN,),
        in_specs=[
            pl.BlockSpec((BN, H), lambda i: (i, 0)),
            pl.BlockSpec((H, H), lambda i: (0, 0)),
            pl.BlockSpec((24, H), lambda i: (0, 0)),
        ],
        out_specs=pl.BlockSpec((BN, H), lambda i: (i, 0)),
        out_shape=jax.ShapeDtypeStruct((NPAD, H), f32),
    )(xp, psel, pos_pad)


def _k1_body(h_ref, w_ref, as_ref, ad_ref, hs_ref, av_ref, dv_ref):
    hb = h_ref[...]
    hsb = jnp.dot(hb, w_ref[0], preferred_element_type=f32)
    hs_ref[0] = hsb
    av_ref[0, 0] = jnp.sum(hsb * as_ref[0], axis=1)
    dv_ref[0, 0] = jnp.sum(hsb * ad_ref[0], axis=1)


def _project(h, W, a_s, a_d):
    return pl.pallas_call(
        _k1_body,
        grid=(NPAD // BN, R),
        in_specs=[
            pl.BlockSpec((BN, H), lambda i, r: (i, 0)),
            pl.BlockSpec((1, H, H), lambda i, r: (r, 0, 0)),
            pl.BlockSpec((1, 1, H), lambda i, r: (r, 0, 0)),
            pl.BlockSpec((1, 1, H), lambda i, r: (r, 0, 0)),
        ],
        out_specs=[
            pl.BlockSpec((1, BN, H), lambda i, r: (r, i, 0)),
            pl.BlockSpec((1, 1, BN), lambda i, r: (r, 0, i)),
            pl.BlockSpec((1, 1, BN), lambda i, r: (r, 0, i)),
        ],
        out_shape=[
            jax.ShapeDtypeStruct((R, NPAD, H), f32),
            jax.ShapeDtypeStruct((R, 1, NPAD), f32),
            jax.ShapeDtypeStruct((R, 1, NPAD), f32),
        ],
    )(h, W, a_s.reshape(R, 1, H), a_d.reshape(R, 1, H))


def _gelu_ln(hb, g_ref, be_ref):
    hb = 0.5 * hb * (1.0 + lax.erf(hb * 0.7071067811865476))
    mu = jnp.mean(hb, axis=1, keepdims=True)
    d = hb - mu
    v = jnp.mean(d * d, axis=1, keepdims=True)
    return d * lax.rsqrt(v + 1e-5) * g_ref[...] + be_ref[...]


def _k2_body(p_ref, bias_ref, g_ref, be_ref, o_ref):
    hb = p_ref[0] + p_ref[1] + bias_ref[...]
    o_ref[...] = _gelu_ln(hb, g_ref, be_ref)


def _k2s_body(p_ref, bias_ref, g_ref, be_ref, q_ref, o_ref, s_ref):
    hb = p_ref[0] + p_ref[1] + bias_ref[...]
    hn = _gelu_ln(hb, g_ref, be_ref)
    o_ref[...] = hn
    s_ref[...] = jnp.sum(hn * q_ref[...], axis=1, keepdims=True)


def _post(parts, bias_sum, g, be, query=None):
    base_in = [
        pl.BlockSpec((2, BN, H), lambda i: (0, i, 0)),
        pl.BlockSpec((1, H), lambda i: (0, 0)),
        pl.BlockSpec((1, H), lambda i: (0, 0)),
        pl.BlockSpec((1, H), lambda i: (0, 0)),
    ]
    if query is None:
        return pl.pallas_call(
            _k2_body,
            grid=(NPAD // BN,),
            in_specs=base_in,
            out_specs=pl.BlockSpec((BN, H), lambda i: (i, 0)),
            out_shape=jax.ShapeDtypeStruct((NPAD, H), f32),
        )(parts, bias_sum, g, be)
    return pl.pallas_call(
        _k2s_body,
        grid=(NPAD // BN,),
        in_specs=base_in + [pl.BlockSpec((1, H), lambda i: (0, 0))],
        out_specs=[
            pl.BlockSpec((BN, H), lambda i: (i, 0)),
            pl.BlockSpec((BN, 1), lambda i: (i, 0)),
        ],
        out_shape=[
            jax.ShapeDtypeStruct((NPAD, H), f32),
            jax.ShapeDtypeStruct((NPAD, 1), f32),
        ],
    )(parts, bias_sum, g, be, query)


def _k3_body(h_ref, b_ref, m_ref, q_ref, wp_ref, bp_ref, o_ref, accn, accd):
    i = pl.program_id(0)

    @pl.when(i == 0)
    def _():
        accn[...] = jnp.zeros_like(accn)
        accd[...] = jnp.zeros_like(accd)

    hb = h_ref[...]
    s2 = jnp.sum(hb * q_ref[...], axis=1, keepdims=True)
    e = jnp.exp(s2 - m_ref[0, 0])
    ohT = (lax.broadcasted_iota(i32, (B, 1), 0) == b_ref[...]).astype(f32)
    accn[...] += jnp.dot(ohT, e * hb, preferred_element_type=f32)
    accd[...] += jnp.dot(ohT, jnp.broadcast_to(e, (BN, H)),
                         preferred_element_type=f32)

    @pl.when(i == NPAD // BN - 1)
    def _():
        pool = accn[...] / (accd[...] + 1e-16)
        o_ref[...] = jnp.dot(pool, wp_ref[...], preferred_element_type=f32) + bp_ref[...]


def _pool(h, batch2d, M, query, Wp, bp):
    return pl.pallas_call(
        _k3_body,
        grid=(NPAD // BN,),
        in_specs=[
            pl.BlockSpec((BN, H), lambda i: (i, 0)),
            pl.BlockSpec((1, BN), lambda i: (0, i)),
            pl.BlockSpec(memory_space=pltpu.SMEM),
            pl.BlockSpec((1, H), lambda i: (0, 0)),
            pl.BlockSpec((H, H), lambda i: (0, 0)),
            pl.BlockSpec((1, H), lambda i: (0, 0)),
        ],
        out_specs=pl.BlockSpec((B, H), lambda i: (0, 0)),
        out_shape=jax.ShapeDtypeStruct((B, H), f32),
        scratch_shapes=[pltpu.VMEM((B, H), f32), pltpu.VMEM((B, H), f32)],
    )(h, batch2d, M, query, Wp, bp)


_MESH = plsc.VectorSubcoreMesh(core_axis_name="c", subcore_axis_name="s",
                               num_cores=2, num_subcores=NTILES)


def _sc_body(edges, hs, av, dv, cvec, out, den_o,
             srcb, dstb, wb, gbuf, gbuf2, praw, cv,
             csrc, cdst, cwb, idxst, pidxA, pidxB, rows, zb1, zb2, sem, sem2,
             den_s, acc_s):
    c = lax.axis_index("c")
    s = lax.axis_index("s")
    zv = jnp.zeros((16,), f32)
    ziv = jnp.zeros((16,), i32)
    iota16 = lax.iota(i32, 16)
    colv = [(q * 16 + iota16) for q in range(8)]
    mo = lambda v: pl.multiple_of(v, 8)

    def _z1(i, _):
        zb1[pl.ds(i * 16, 16)] = zv
        return 0
    lax.fori_loop(0, 800 // 16, _z1, 0)

    def _z2(i, _):
        for q in range(8):
            zb2[i, pl.ds(q * 16, 16)] = zv
        return 0
    lax.fori_loop(0, 40, _z2, 0)

    for j in range(3):
        r = c * 3 + j
        pltpu.sync_copy(edges.at[pl.ds(mo(((r * 2 + 0) * NTILES + s) * TE), TE)],
                        srcb.at[pl.ds(j * TE, TE)])
        pltpu.sync_copy(edges.at[pl.ds(mo(((r * 2 + 1) * NTILES + s) * TE), TE)],
                        dstb.at[pl.ds(j * TE, TE)])
        off = r * NPAD

        def _adj(i, _, j=j, off=off):
            sl = pl.ds(j * TE + i * 16, 16)
            srcb[sl] = srcb[sl] + off
            dstb[sl] = dstb[sl] + off
            return 0
        lax.fori_loop(0, TE // 16, _adj, 0)

    NSL = NPAD // NTILES

    for j in range(3):
        r = c * 3 + j
        def _zd(i, _):
            pltpu.sync_copy(zb1, den_s.at[pl.ds(mo(s * NSL + i * 800), 800)])
            return 0
        lax.fori_loop(0, 4, _zd, 0)
        plsc.subcore_barrier()
        pltpu.sync_copy(cvec.at[r], cv)
        cvv = cv[...]

        def _batch_a(b, _, j=j, r=r, cvv=cvv):
            base = j * TE + b * 896

            def _st1(k, _):
                idxst[0, pl.ds(k * 16, 16)] = srcb[pl.ds(base + k * 16, 16)]
                return 0
            lax.fori_loop(0, 56, _st1, 0)
            cpa = pltpu.async_copy(av.at[idxst.at[0]], gbuf, sem)

            def _st2(k, _):
                praw[0, pl.ds(k * 16, 16)] = dstb[pl.ds(base + k * 16, 16)]
                return 0
            lax.fori_loop(0, 56, _st2, 0)
            cpb = pltpu.async_copy(dv.at[praw.at[0]], gbuf2, sem2)
            cpa.wait()
            cpb.wait()

            def _cmp(k, _, cvv=cvv):
                slk = pl.ds(k * 16, 16)
                al = gbuf[slk] + gbuf2[slk]
                al = jnp.where(al > 0, al, 0.2 * al)
                ev = jnp.exp(al - cvv)
                vid = b * 896 + k * 16 + iota16
                ev = jnp.where(vid < EPT, ev, 0.0)
                wb[pl.ds(base + k * 16, 16)] = ev
                praw[0, slk] = praw[0, slk] - r * NPAD
                return 0
            lax.fori_loop(0, 56, _cmp, 0)
            pltpu.sync_copy(wb.at[pl.ds(pl.multiple_of(base, 128), 896)],
                            den_s.at[praw.at[0]], add=True)
            return 0
        lax.fori_loop(0, TE // 896, _batch_a, 0)
        plsc.subcore_barrier()
        pltpu.sync_copy(den_s.at[pl.ds(mo(s * NSL), NSL)],
                        den_o.at[pl.ds(mo(r * NPAD + s * NSL), NSL)])
        plsc.subcore_barrier()

        def _batch_w(b, _, j=j):
            base = j * TE + b * 896

            def _st3(k, _):
                idxst[0, pl.ds(k * 16, 16)] = dstb[pl.ds(base + k * 16, 16)]
                return 0
            lax.fori_loop(0, 56, _st3, 0)
            pltpu.async_copy(den_o.at[idxst.at[0]], gbuf, sem).wait()

            def _div(k, _):
                sl = pl.ds(base + k * 16, 16)
                wb[sl] = wb[sl] / (gbuf[pl.ds(k * 16, 16)] + 1e-16)
                return 0
            lax.fori_loop(0, 56, _div, 0)
            return 0
        lax.fori_loop(0, TE // 896, _batch_w, 0)

    def _chunk(ch, _):
        lo = ch * CHUNK

        def _zacc(i, _):
            pltpu.sync_copy(zb2, acc_s.at[pl.ds(mo(s * RPT + i * 40), 40), :])
            return 0
        lax.fori_loop(0, RPT // 40, _zacc, 0)
        plsc.subcore_barrier()

        def _rel(j2, _, lo=lo):
            r = c * 3 + j2
            cbase = r * NPAD + lo

            def _scan(b, ptr, j2=j2, cbase=cbase):
                for k in range(8):
                    sl = pl.ds(j2 * TE + b * 128 + k * 16, 16)
                    d = dstb[sl]
                    m = (d >= cbase) & (d < cbase + CHUNK)
                    vid = b * 128 + k * 16 + iota16
                    m = m & (vid < EPT)
                    pos = plsc.cumsum(m.astype(i32))
                    off = jnp.where(m, ptr + pos - 1, DUMP + iota16)
                    plsc.store_scatter(cdst, [off], d - cbase)
                    plsc.store_scatter(csrc, [off], srcb[sl])
                    plsc.store_scatter(cwb, [off], wb[sl])
                    ptr = ptr + pos[15]
                return ptr
            ptr = lax.fori_loop(0, NB_A, _scan, jnp.int32(0))

            for k in range(8):
                psl = pl.ds(ptr + k * 16, 16)
                cdst[psl] = ziv
                csrc[psl] = ziv
                cwb[psl] = zv
            nb2 = (ptr + 63) // 64

            def _gbody(g, myp, otp, myoff, otoff, mysem, otsem, nb2=nb2):
                half = rows.at[pl.ds(myoff, 64), :]
                pltpu.make_async_copy(hs.at[myp.at[0]], half, mysem).wait()

                @pl.when(g + 1 < nb2)
                def _():
                    for k in range(4):
                        otp[0, pl.ds(k * 16, 16)] = (
                            csrc[pl.ds((g + 1) * 64 + k * 16, 16)])
                    pltpu.async_copy(hs.at[otp.at[0]],
                                     rows.at[pl.ds(otoff, 64), :], otsem)
                for k in range(4):
                    myp[0, pl.ds(k * 16, 16)] = cdst[pl.ds(g * 64 + k * 16, 16)]

                def _scale(e2, _):
                    rsp = jnp.full((16,), myoff + e2, i32)
                    wsp = plsc.load_gather(cwb, [jnp.full((16,), g * 64 + e2, i32)])
                    for q in range(8):
                        v = plsc.load_gather(rows, [rsp, colv[q]])
                        plsc.store_scatter(rows, [rsp, colv[q]], v * wsp)
                    return 0
                lax.fori_loop(0, 64, _scale, 0)
                pltpu.sync_copy(half, acc_s.at[myp.at[0]], add=True)

            @pl.when(nb2 > 0)
            def _():
                for k in range(4):
                    pidxA[0, pl.ds(k * 16, 16)] = csrc[pl.ds(k * 16, 16)]
                pltpu.async_copy(hs.at[pidxA.at[0]], rows.at[pl.ds(0, 64), :],
                                 sem)

            def _gsc(g, _):
                @pl.when(lax.rem(g, 2) == 0)
                def _():
                    _gbody(g, pidxA, pidxB, 0, 64, sem, sem2)

                @pl.when(lax.rem(g, 2) == 1)
                def _():
                    _gbody(g, pidxB, pidxA, 64, 0, sem2, sem)
                return 0
            lax.fori_loop(0, nb2, _gsc, 0)
            return 0
        lax.fori_loop(0, 3, _rel, 0)
        plsc.subcore_barrier()

        pltpu.sync_copy(acc_s.at[pl.ds(mo(s * RPT), RPT), :],
                        out.at[c, pl.ds(mo(lo + s * RPT), RPT), :])
        plsc.subcore_barrier()
        return 0
    lax.fori_loop(0, NCHUNK, _chunk, 0)


@functools.partial(
    pl.kernel,
    out_type=(jax.ShapeDtypeStruct((2, NPAD, H), f32),
              jax.ShapeDtypeStruct((R * NPAD,), f32)),
    mesh=_MESH,
    compiler_params=pltpu.CompilerParams(needs_layout_passes=False),
    scratch_types=[
        pltpu.VMEM((3 * TE,), i32),
        pltpu.VMEM((3 * TE,), i32),
        pltpu.VMEM((3 * TE,), f32),
        pltpu.VMEM((896,), f32),
        pltpu.VMEM((896,), f32),
        pltpu.VMEM((1, 896), i32),
        pltpu.VMEM((16,), f32),
        pltpu.VMEM((CAP,), i32),
        pltpu.VMEM((CAP,), i32),
        pltpu.VMEM((CAP,), f32),
        pltpu.VMEM((1, 896), i32),
        pltpu.VMEM((1, 64), i32),
        pltpu.VMEM((1, 64), i32),
        pltpu.VMEM((128, H), f32),
        pltpu.VMEM((800,), f32),
        pltpu.VMEM((40, H), f32),
        pltpu.SemaphoreType.DMA,
        pltpu.SemaphoreType.DMA,
        pltpu.VMEM_SHARED((NPAD,), f32),
        pltpu.VMEM_SHARED((CHUNK, H), f32),
    ],
)
def _sc_edge(edges, hs, av, dv, cvec, out, den_o, *scratch):
    _sc_body(edges, hs, av, dv, cvec, out, den_o, *scratch)


def _gat_layer(h, edges_t, W, a_s, a_d):
    hs, av, dv = _project(h, W, a_s, a_d)
    av2 = av.reshape(R, NPAD)
    dv2 = dv.reshape(R, NPAD)
    z = av2.max(axis=1) + dv2.max(axis=1)
    Cr = jnp.where(z > 0, z, 0.2 * z)
    cvec = jnp.tile(Cr[:, None], (1, 16))
    parts, _ = _sc_edge(edges_t.reshape(-1), hs.reshape(R * NPAD, H),
                        av2.reshape(-1), dv2.reshape(-1), cvec)
    return parts


_SEL = np.zeros((H, H), np.float32)
for _j in range(120):
    _SEL[_j if _j < 5 else _j + 1, _j] = 1.0


def kernel(x, edge_index, batch, pos_emb, W1, a_src1, a_dst1, b1,
           W2, a_src2, a_dst2, b2, g1, be1, g2, be2, query, Wp, bp):
    xp = jnp.zeros((NPAD, H), f32).at[:N, :IN].set(x)
    batch2d = jnp.full((1, NPAD), B, i32).at[0, :N].set(batch)
    edges_t = (jnp.zeros((R, 2, NTILES, TE), i32)
               .at[:, :, :, :EPT].set(edge_index.reshape(R, 2, NTILES, EPT)))
    psel = jnp.asarray(_SEL)
    pos_pad = jnp.zeros((24, H), f32).at[:, 120:].set(pos_emb)
    g1r = g1.reshape(1, H)
    be1r = be1.reshape(1, H)
    g2r = g2.reshape(1, H)
    be2r = be2.reshape(1, H)
    qr = query.reshape(1, H)
    bpr = bp.reshape(1, H)
    bs1 = b1.sum(axis=0).reshape(1, H)
    bs2 = b2.sum(axis=0).reshape(1, H)

    h = _assemble(xp, psel, pos_pad)
    parts1 = _gat_layer(h, edges_t, W1, a_src1, a_dst1)
    h = _post(parts1, bs1, g1r, be1r)
    parts2 = _gat_layer(h, edges_t, W2, a_src2, a_dst2)
    h, scores = _post(parts2, bs2, g2r, be2r, query=qr)
    M = jnp.max(scores).reshape(1, 1)
    return _pool(h, batch2d, M, qr, Wp, bpr)
